# baseline (speedup 1.0000x reference)
import functools
import math

import jax
import jax.numpy as jnp
from jax import lax
from jax.experimental import pallas as pl
from jax.experimental.pallas import tpu as pltpu
from jax.experimental.pallas import tpu_sc as plsc

L = 16


@functools.lru_cache(maxsize=None)
def _make_sc_gather(R, S, V, D, C):
    info = plsc.get_sparse_core_info()
    NC, NS = info.num_cores, info.num_subcores
    NW = NC * NS
    B = R * S
    assert B % NW == 0
    b_per_w = B // NW
    w_per_row = NW // R
    assert S % w_per_row == 0
    n_chunks = b_per_w // C
    NBUF = 3
    scale = jnp.float32(math.sqrt(D))

    mesh = plsc.VectorSubcoreMesh(core_axis_name="c", subcore_axis_name="s")

    @functools.partial(
        pl.kernel,
        mesh=mesh,
        out_type=jax.ShapeDtypeStruct((B, D), jnp.float32),
        scratch_types=[
            pltpu.VMEM((b_per_w,), jnp.int32),
        ]
        + [pltpu.VMEM((C, D), jnp.float32)] * NBUF
        + [pltpu.SemaphoreType.DMA] * (2 * NBUF),
    )
    def k(idx_hbm, table_hbm, out_hbm, idx_v, *scratch):
        bufs = scratch[:NBUF]
        gsems = scratch[NBUF : 2 * NBUF]
        ssems = scratch[2 * NBUF : 3 * NBUF]
        wid = lax.axis_index("s") * NC + lax.axis_index("c")
        base = wid * b_per_w
        pltpu.sync_copy(idx_hbm.at[pl.ds(base, b_per_w)], idx_v)

        def start_gather(g):
            b = g % NBUF
            pltpu.async_copy(
                table_hbm.at[idx_v.at[pl.ds(g * C, C)]], bufs[b], gsems[b]
            )

        def wait_gather(g):
            b = g % NBUF
            pltpu.make_async_copy(
                table_hbm.at[idx_v.at[pl.ds(g * C, C)]], bufs[b], gsems[b]
            ).wait()

        def start_scatter(g):
            b = g % NBUF
            pltpu.async_copy(
                bufs[b], out_hbm.at[pl.ds(base + g * C, C)], ssems[b]
            )

        def wait_scatter(g):
            b = g % NBUF
            pltpu.make_async_copy(
                bufs[b], out_hbm.at[pl.ds(base + g * C, C)], ssems[b]
            ).wait()

        def scale_buf(buf):
            def row_body(r, _):
                for rr in range(2):
                    for j in range(D // L):
                        sl = pl.ds(j * L, L)
                        buf[2 * r + rr, sl] = buf[2 * r + rr, sl] * scale
                return 0

            lax.fori_loop(0, C // 2, row_body, 0)

        for g in range(NBUF):
            start_gather(g)
        for g in range(n_chunks):
            wait_gather(g)
            nxt = g + 1
            if NBUF - 1 <= g < n_chunks - 1:
                wait_scatter(nxt - NBUF)
                start_gather(nxt)
            scale_buf(bufs[g % NBUF])
            start_scatter(g)
        for g in range(n_chunks - NBUF, n_chunks):
            wait_scatter(g)

    return k


def kernel(token_ids, W):
    R, S = token_ids.shape
    V, D = W.shape
    idx = token_ids.reshape(R * S).astype(jnp.int32)
    out = _make_sc_gather(R, S, V, D, 32)(idx, W)
    return out.reshape(R, S, D)


# speedup vs baseline: 1.1623x; 1.1623x over previous
import functools
import math

import jax
import jax.numpy as jnp
from jax import lax
from jax.experimental import pallas as pl
from jax.experimental.pallas import tpu as pltpu
from jax.experimental.pallas import tpu_sc as plsc

L = 16


@functools.lru_cache(maxsize=None)
def _make_sc_gather(R, S, V, D, C):
    info = plsc.get_sparse_core_info()
    NC, NS = info.num_cores, info.num_subcores
    NW = NC * NS
    B = R * S
    assert B % NW == 0
    b_per_w = B // NW
    w_per_row = NW // R
    assert S % w_per_row == 0
    n_chunks = b_per_w // C
    NBUF = 3
    scale = jnp.float32(math.sqrt(D))

    mesh = plsc.VectorSubcoreMesh(core_axis_name="c", subcore_axis_name="s")

    @functools.partial(
        pl.kernel,
        mesh=mesh,
        out_type=jax.ShapeDtypeStruct((B, D), jnp.float32),
        scratch_types=[
            pltpu.VMEM((b_per_w,), jnp.int32),
        ]
        + [pltpu.VMEM((C, D), jnp.float32)] * NBUF
        + [pltpu.SemaphoreType.DMA] * (2 * NBUF),
    )
    def k(idx_hbm, table_hbm, out_hbm, idx_v, *scratch):
        bufs = scratch[:NBUF]
        gsems = scratch[NBUF : 2 * NBUF]
        ssems = scratch[2 * NBUF : 3 * NBUF]
        wid = lax.axis_index("s") * NC + lax.axis_index("c")
        base = wid * b_per_w
        pltpu.sync_copy(idx_hbm.at[pl.ds(base, b_per_w)], idx_v)

        def start_gather(g):
            b = g % NBUF
            pltpu.async_copy(
                table_hbm.at[idx_v.at[pl.ds(g * C, C)]], bufs[b], gsems[b]
            )

        def wait_gather(g):
            b = g % NBUF
            pltpu.make_async_copy(
                table_hbm.at[idx_v.at[pl.ds(g * C, C)]], bufs[b], gsems[b]
            ).wait()

        def start_scatter(g):
            b = g % NBUF
            pltpu.async_copy(
                bufs[b], out_hbm.at[pl.ds(base + g * C, C)], ssems[b]
            )

        def wait_scatter(g):
            b = g % NBUF
            pltpu.make_async_copy(
                bufs[b], out_hbm.at[pl.ds(base + g * C, C)], ssems[b]
            ).wait()

        def scale_buf(buf):
            def row_body(r, _):
                for j in range(D // L):
                    sl = pl.ds(j * L, L)
                    buf[r, sl] = buf[r, sl] * scale
                return 0

            lax.fori_loop(0, C, row_body, 0)

        start_gather(0)
        start_gather(1)
        for g in range(n_chunks):
            wait_gather(g)
            if g >= 1:
                start_scatter(g - 1)
            if g >= 2:
                wait_scatter(g - 2)
            if g + 1 < n_chunks and g >= 1:
                start_gather(g + 1)
            scale_buf(bufs[g % NBUF])
        start_scatter(n_chunks - 1)
        wait_scatter(n_chunks - 2)
        wait_scatter(n_chunks - 1)

    return k


def kernel(token_ids, W):
    R, S = token_ids.shape
    V, D = W.shape
    idx = token_ids.reshape(R * S).astype(jnp.int32)
    out = _make_sc_gather(R, S, V, D, 32)(idx, W)
    return out.reshape(R, S, D)
